# Initial kernel scaffold; baseline (speedup 1.0000x reference)
import jax
import jax.numpy as jnp
from jax.experimental import pallas as pl


def kernel(x, table):
    raise NotImplementedError("write your pallas kernel here")


# speedup vs baseline: 1.8393x; 1.8393x over previous
import functools
import math

import numpy as np
import jax
import jax.numpy as jnp
from jax import lax
from jax.experimental import pallas as pl
from jax.experimental.pallas import tpu as pltpu
from jax.experimental.pallas import tpu_sc as plsc

D_MODEL = 768
MAX_SEQ_LEN = 2048
_SCALE = math.sqrt(float(D_MODEL))
_LANES = 16


def _pe_host() -> np.ndarray:
    pos = np.arange(MAX_SEQ_LEN, dtype=np.float64).reshape(-1, 1)
    i = np.arange(D_MODEL, dtype=np.float64)
    rads = pos / np.power(10000.0, 2.0 * np.floor(i / 2.0) / D_MODEL)
    pe = np.zeros((MAX_SEQ_LEN, D_MODEL), dtype=np.float32)
    pe[:, 0::2] = np.sin(rads[:, 0::2]).astype(np.float32)
    pe[:, 1::2] = np.cos(rads[:, 1::2]).astype(np.float32)
    return pe


_PE = _pe_host()


@functools.lru_cache(maxsize=None)
def _build(n_rows: int):
    info = plsc.get_sparse_core_info()
    nc, ns = info.num_cores, info.num_subcores
    nw = nc * ns
    rpw = n_rows // nw
    chunk = 64
    nchunk = rpw // chunk
    groups = D_MODEL // _LANES

    mesh = plsc.VectorSubcoreMesh(core_axis_name="c", subcore_axis_name="s")

    @functools.partial(
        pl.kernel,
        mesh=mesh,
        out_type=jax.ShapeDtypeStruct((n_rows, D_MODEL), jnp.float32),
        scratch_types=[
            pltpu.VMEM((nchunk, chunk), jnp.int32),
            pltpu.VMEM((chunk, D_MODEL), jnp.float32),
            pltpu.VMEM((chunk, D_MODEL), jnp.float32),
            pltpu.SemaphoreType.DMA,
        ],
    )
    def emb(x_hbm, table_hbm, pe_hbm, out_hbm, idx_v, tok_v, pe_v, sem):
        wid = lax.axis_index("s") * nc + lax.axis_index("c")
        base = wid * rpw
        pos_base = lax.rem(base, MAX_SEQ_LEN)
        pltpu.sync_copy(x_hbm.at[wid], idx_v)
        for c in range(nchunk):
            pltpu.async_copy(table_hbm.at[idx_v.at[c]], tok_v, sem).wait()
            pltpu.sync_copy(pe_hbm.at[pl.ds(pos_base + c * chunk, chunk)], pe_v)

            def row_body(r, _):
                for g in range(groups):
                    sl = pl.ds(g * _LANES, _LANES)
                    tok_v[r, sl] = tok_v[r, sl] * _SCALE + pe_v[r, sl]
                return 0

            lax.fori_loop(0, chunk, row_body, 0)
            pltpu.sync_copy(tok_v, out_hbm.at[pl.ds(base + c * chunk, chunk)])

    return emb, nw, nchunk, chunk


def kernel(x, table):
    b, s = x.shape
    n_rows = b * s
    emb, nw, nchunk, chunk = _build(n_rows)
    x3 = x.reshape(nw, nchunk, chunk).astype(jnp.int32)
    pe = jnp.asarray(_PE)
    out = emb(x3, table, pe)
    return out.reshape(b, s, D_MODEL)
